# baseline (speedup 1.0000x reference)
import functools

import jax
import jax.numpy as jnp
from jax import lax
from jax.experimental import pallas as pl
from jax.experimental.pallas import tpu as pltpu
from jax.experimental.pallas import tpu_sc as plsc

N = 10000
E = 320000
D = 128
K = 10
OC = 128

NC = 2
NS = 16
NW = NC * NS
N_PAD = 10240
SLAB = N_PAD // NS
CW = 128
NCH = 80
EPW = E // NW
TRASH = N_PAD - 1

F32 = jnp.float32
I32 = jnp.int32


def _sc_mesh():
    return plsc.VectorSubcoreMesh(core_axis_name="c", subcore_axis_name="s")


def _sc_degrees(es_p):
    @functools.partial(
        pl.kernel,
        out_type=jax.ShapeDtypeStruct((NC, 2, N_PAD), F32),
        mesh=_sc_mesh(),
        scratch_types=[
            pltpu.VMEM((NCH, CW), I32),
            pltpu.VMEM((NCH, CW), I32),
            pltpu.VMEM((SLAB,), F32),
            pltpu.VMEM((CW,), F32),
            pltpu.VMEM_SHARED((N_PAD,), F32),
            pltpu.VMEM_SHARED((N_PAD,), F32),
        ],
    )
    def kern(es_hbm, out_hbm, src_v, dst_v, z_v, ones_v, d_src, d_dst):
        c = lax.axis_index("c")
        s = lax.axis_index("s")
        wid = s * NC + c
        pltpu.sync_copy(es_hbm.at[0, wid], src_v)
        pltpu.sync_copy(es_hbm.at[1, wid], dst_v)

        def zfill(i, _):
            z_v[pl.ds(i * 16, 16)] = jnp.zeros((16,), F32)
            return ()

        lax.fori_loop(0, SLAB // 16, zfill, ())
        def ofill(i, _):
            ones_v[pl.ds(i * 16, 16)] = jnp.ones((16,), F32)
            return ()

        lax.fori_loop(0, CW // 16, ofill, ())
        pltpu.sync_copy(z_v, d_src.at[pl.ds(s * SLAB, SLAB)])
        pltpu.sync_copy(z_v, d_dst.at[pl.ds(s * SLAB, SLAB)])
        plsc.subcore_barrier()

        def chunk(i, _):
            pltpu.sync_copy(ones_v, d_src.at[src_v.at[i]], add=True)
            pltpu.sync_copy(ones_v, d_dst.at[dst_v.at[i]], add=True)
            return ()

        lax.fori_loop(0, NCH, chunk, ())
        plsc.subcore_barrier()
        pltpu.sync_copy(d_src.at[pl.ds(s * SLAB, SLAB)],
                        out_hbm.at[c, 0, pl.ds(s * SLAB, SLAB)])
        pltpu.sync_copy(d_dst.at[pl.ds(s * SLAB, SLAB)],
                        out_hbm.at[c, 1, pl.ds(s * SLAB, SLAB)])

    return kern(es_p)


def _sc_edges(y, es_p):
    @functools.partial(
        pl.kernel,
        out_type=jax.ShapeDtypeStruct((NC, N_PAD, D), F32),
        mesh=_sc_mesh(),
        scratch_types=[
            pltpu.VMEM((NCH, CW), I32),
            pltpu.VMEM((NCH, CW), I32),
            pltpu.VMEM((CW, D), F32),
            pltpu.VMEM_SHARED((N_PAD, D), F32),
            pltpu.SemaphoreType.DMA,
        ],
    )
    def kern(y_hbm, es_hbm, out_hbm, src_v, dst_v, buf, acc, rsem):
        c = lax.axis_index("c")
        s = lax.axis_index("s")
        wid = s * NC + c
        pltpu.sync_copy(es_hbm.at[0, wid], src_v)
        pltpu.sync_copy(es_hbm.at[1, wid], dst_v)

        zb = 64

        def zfill(i, _):
            for q in range(D // 16):
                buf[i, pl.ds(q * 16, 16)] = jnp.zeros((16,), F32)
            return ()

        lax.fori_loop(0, zb, zfill, ())

        def zslab(t, _):
            pltpu.sync_copy(buf.at[pl.ds(0, zb)],
                            acc.at[pl.ds(s * SLAB + t * zb, zb)])
            return ()

        lax.fori_loop(0, SLAB // zb, zslab, ())
        plsc.subcore_barrier()

        def step(i, _):
            pltpu.async_copy(y_hbm.at[src_v.at[i]], buf, rsem).wait()
            pltpu.sync_copy(buf, acc.at[dst_v.at[i]], add=True)
            return ()

        lax.fori_loop(0, NCH, step, ())
        plsc.subcore_barrier()
        pltpu.sync_copy(acc.at[pl.ds(s * SLAB, SLAB)],
                        out_hbm.at[c, pl.ds(s * SLAB, SLAB)])

    return kern(y, es_p)


def _prep_kern(deg_ref, f_ref, y0_ref, ns_ref, nd_ref):
    d_src = deg_ref[0, 0] + deg_ref[1, 0]
    d_dst = deg_ref[0, 1] + deg_ref[1, 1]
    ns = jnp.where(d_src > 0, lax.rsqrt(d_src), 0.0)
    nd = jnp.where(d_dst > 0, lax.rsqrt(d_dst), 0.0)
    y0_ref[...] = f_ref[...] * ns
    ns_ref[...] = ns
    nd_ref[...] = nd


def _tc_prep(feats_p, degparts):
    deg4 = degparts.reshape(NC, 2, N_PAD, 1)
    grid = N_PAD // 128
    return pl.pallas_call(
        _prep_kern,
        grid=(grid,),
        in_specs=[
            pl.BlockSpec((NC, 2, 128, 1), lambda r: (0, 0, r, 0)),
            pl.BlockSpec((128, D), lambda r: (r, 0)),
        ],
        out_specs=[
            pl.BlockSpec((128, D), lambda r: (r, 0)),
            pl.BlockSpec((128, 1), lambda r: (r, 0)),
            pl.BlockSpec((128, 1), lambda r: (r, 0)),
        ],
        out_shape=[
            jax.ShapeDtypeStruct((N_PAD, D), F32),
            jax.ShapeDtypeStruct((N_PAD, 1), F32),
            jax.ShapeDtypeStruct((N_PAD, 1), F32),
        ],
    )(deg4, feats_p)


def _dense1_kern(agg_ref, nd_ref, w_ref, b_ref, ns_ref, out_ref):
    agg = agg_ref[0] + agg_ref[1]
    x = agg * nd_ref[...]
    h = jnp.dot(x, w_ref[...], preferred_element_type=F32) + b_ref[...]
    out_ref[...] = jnp.maximum(h, 0.0) * ns_ref[...]


def _tc_dense1(aggparts, nd, w, b2d, ns):
    grid = N_PAD // 128
    return pl.pallas_call(
        _dense1_kern,
        grid=(grid,),
        in_specs=[
            pl.BlockSpec((NC, 128, D), lambda r: (0, r, 0)),
            pl.BlockSpec((128, 1), lambda r: (r, 0)),
            pl.BlockSpec((D, D), lambda r: (0, 0)),
            pl.BlockSpec((1, D), lambda r: (0, 0)),
            pl.BlockSpec((128, 1), lambda r: (r, 0)),
        ],
        out_specs=pl.BlockSpec((128, D), lambda r: (r, 0)),
        out_shape=jax.ShapeDtypeStruct((N_PAD, D), F32),
    )(aggparts, nd, w, b2d, ns)


def _dense2_kern(agg_ref, nd_ref, w_ref, b_ref, h_ref, rm_ref):
    agg = agg_ref[0] + agg_ref[1]
    x = agg * nd_ref[...]
    h = jnp.dot(x, w_ref[...], preferred_element_type=F32) + b_ref[...]
    h = jnp.maximum(h, 0.0)
    h_ref[...] = h
    rm = jnp.max(h, axis=1, keepdims=True)
    ridx = pl.program_id(0) * 128 + lax.broadcasted_iota(I32, (128, 1), 0)
    rm_ref[...] = jnp.where(ridx < N, rm, -1.0)


def _tc_dense2(aggparts, nd, w, b2d):
    grid = N_PAD // 128
    return pl.pallas_call(
        _dense2_kern,
        grid=(grid,),
        in_specs=[
            pl.BlockSpec((NC, 128, D), lambda r: (0, r, 0)),
            pl.BlockSpec((128, 1), lambda r: (r, 0)),
            pl.BlockSpec((D, D), lambda r: (0, 0)),
            pl.BlockSpec((1, D), lambda r: (0, 0)),
        ],
        out_specs=[
            pl.BlockSpec((128, D), lambda r: (r, 0)),
            pl.BlockSpec((128, 1), lambda r: (r, 0)),
        ],
        out_shape=[
            jax.ShapeDtypeStruct((N_PAD, D), F32),
            jax.ShapeDtypeStruct((N_PAD, 1), F32),
        ],
    )(aggparts, nd, w, b2d)


def _topk_kern(rm_ref, idx_ref):
    vals0 = rm_ref[...]
    lin = (lax.broadcasted_iota(I32, (80, 128), 0) * 128
           + lax.broadcasted_iota(I32, (80, 128), 1))
    lane = lax.broadcasted_iota(I32, (1, 128), 1)
    big = jnp.int32(2 ** 30)

    def body(k, carry):
        vals, acc = carry
        m = jnp.max(vals)
        cand = jnp.where(vals == m, lin, big)
        j = jnp.min(cand)
        acc = jnp.where(lane == k, j, acc)
        vals = jnp.where(lin == j, -2.0, vals)
        return vals, acc

    _, acc = lax.fori_loop(0, K, body, (vals0, jnp.zeros((1, 128), I32)))
    idx_ref[...] = acc


def _tc_topk(rm2d):
    return pl.pallas_call(
        _topk_kern,
        out_shape=jax.ShapeDtypeStruct((1, 128), I32),
    )(rm2d)


def _gather_kern(idx_ref, h_ref, o_ref):
    o_ref[...] = h_ref[...]


def _tc_gather(idx10, h):
    h3 = h.reshape(N_PAD, 1, D)
    grid_spec = pltpu.PrefetchScalarGridSpec(
        num_scalar_prefetch=1,
        grid=(K,),
        in_specs=[pl.BlockSpec((1, 1, D), lambda k, idx: (idx[k], 0, 0))],
        out_specs=pl.BlockSpec((1, 1, D), lambda k, idx: (k, 0, 0)),
    )
    return pl.pallas_call(
        _gather_kern,
        grid_spec=grid_spec,
        out_shape=jax.ShapeDtypeStruct((K, 1, D), F32),
    )(idx10, h3).reshape(K, D)


def _tail_kern(p_ref, w1_ref, b1_ref, w2_ref, b2_ref, out_ref):
    p = p_ref[...]
    row = lax.broadcasted_iota(I32, (D, D), 0)
    col = lax.broadcasted_iota(I32, (D, D), 1)
    lane = lax.broadcasted_iota(I32, (1, D), 1)
    for k in [2, 4, 8, 16, 32, 64, 128]:
        j = k // 2
        while j >= 1:
            perm_m = (row == jnp.bitwise_xor(col, j)).astype(F32)
            part = jnp.dot(p, perm_m, preferred_element_type=F32,
                           precision=lax.Precision.HIGHEST)
            take_min = ((lane & k) == 0) ^ ((lane & j) != 0)
            p = jnp.where(take_min, jnp.minimum(p, part), jnp.maximum(p, part))
            j //= 2
    u = jnp.dot(p, w1_ref[...], preferred_element_type=F32,
                precision=lax.Precision.HIGHEST) + b1_ref[...]
    u = jnp.maximum(u, 0.0)
    ar = lax.broadcasted_iota(I32, (8, 16), 0)
    ac = lax.broadcasted_iota(I32, (8, 16), 1)
    a_m = (ac == 2 * ar).astype(F32)
    b_m = (ac == 2 * ar + 1).astype(F32)
    v = jnp.maximum(
        jnp.dot(a_m, u, preferred_element_type=F32, precision=lax.Precision.HIGHEST),
        jnp.dot(b_m, u, preferred_element_type=F32, precision=lax.Precision.HIGHEST),
    )
    acc = b2_ref[...]
    for t in range(5):
        acc = acc + jnp.dot(v[t:t + 1, :], w2_ref[t],
                            preferred_element_type=F32,
                            precision=lax.Precision.HIGHEST)
    out_ref[...] = jnp.maximum(acc, 0.0)


def _tc_tail(pooled_p, cw1r, cb1r, cw2m, cb2r):
    return pl.pallas_call(
        _tail_kern,
        out_shape=jax.ShapeDtypeStruct((1, OC), F32),
    )(pooled_p, cw1r, cb1r, cw2m, cb2r)


def kernel(feats, edge_index, W1, b1, W2, b2, cw1, cb1, cw2, cb2):
    pad = NCH * CW - EPW
    es = edge_index.astype(I32).reshape(2, NW, EPW)
    es_p = jnp.concatenate(
        [es, jnp.full((2, NW, pad), TRASH, I32)], axis=2
    ).reshape(2, NW, NCH, CW)
    feats_p = jnp.pad(feats, ((0, N_PAD - N), (0, 0)))

    degparts = _sc_degrees(es_p)
    y0, ns, nd = _tc_prep(feats_p, degparts)

    b1r = b1.reshape(1, D)
    b2r = b2.reshape(1, D)
    agg1 = _sc_edges(y0, es_p)
    y1 = _tc_dense1(agg1, nd, W1, b1r, ns)
    agg2 = _sc_edges(y1, es_p)
    h2, rm = _tc_dense2(agg2, nd, W2, b2r)

    idxv = _tc_topk(rm.reshape(N_PAD // 128, 128))
    idx10 = idxv[0, :K]
    pooled = _tc_gather(idx10, h2)
    pooled_p = jnp.pad(pooled, ((0, 6), (0, 0)))

    cw1r = jnp.transpose(cw1.reshape(OC // 2, D))
    cb1r = cb1.reshape(1, OC // 2)
    cw2m = jnp.transpose(cw2, (2, 1, 0))
    cb2r = cb2.reshape(1, OC)
    return _tc_tail(pooled_p, cw1r, cb1r, cw2m, cb2r)


# speedup vs baseline: 1.3673x; 1.3673x over previous
import functools

import jax
import jax.numpy as jnp
from jax import lax
from jax.experimental import pallas as pl
from jax.experimental.pallas import tpu as pltpu
from jax.experimental.pallas import tpu_sc as plsc

N = 10000
E = 320000
D = 128
K = 10
OC = 128

NC = 2
NS = 16
NW = NC * NS
N_PAD = 10240
SLAB = N_PAD // NS
CW = 128
NCH = 79
EPW = E // NW
TRASH = N_PAD - 1

F32 = jnp.float32
I32 = jnp.int32


def _sc_mesh():
    return plsc.VectorSubcoreMesh(core_axis_name="c", subcore_axis_name="s")


def _sc_degrees(src_p, dst_p):
    @functools.partial(
        pl.kernel,
        out_type=jax.ShapeDtypeStruct((NC, 2, N_PAD), F32),
        mesh=_sc_mesh(),
        scratch_types=[
            pltpu.VMEM((NCH, CW), I32),
            pltpu.VMEM((NCH, CW), I32),
            pltpu.VMEM((SLAB,), F32),
            pltpu.VMEM((CW,), F32),
            pltpu.VMEM_SHARED((N_PAD,), F32),
            pltpu.VMEM_SHARED((N_PAD,), F32),
        ],
    )
    def kern(src_hbm, dst_hbm, out_hbm, src_v, dst_v, z_v, ones_v, d_src, d_dst):
        c = lax.axis_index("c")
        s = lax.axis_index("s")
        wid = s * NC + c
        pltpu.sync_copy(src_hbm.at[wid], src_v)
        pltpu.sync_copy(dst_hbm.at[wid], dst_v)

        def zfill(i, _):
            z_v[pl.ds(i * 16, 16)] = jnp.zeros((16,), F32)
            return ()

        lax.fori_loop(0, SLAB // 16, zfill, ())
        def ofill(i, _):
            ones_v[pl.ds(i * 16, 16)] = jnp.ones((16,), F32)
            return ()

        lax.fori_loop(0, CW // 16, ofill, ())
        pltpu.sync_copy(z_v, d_src.at[pl.ds(s * SLAB, SLAB)])
        pltpu.sync_copy(z_v, d_dst.at[pl.ds(s * SLAB, SLAB)])
        plsc.subcore_barrier()

        def chunk(i, _):
            pltpu.sync_copy(ones_v, d_src.at[src_v.at[i]], add=True)
            pltpu.sync_copy(ones_v, d_dst.at[dst_v.at[i]], add=True)
            return ()

        lax.fori_loop(0, NCH, chunk, ())
        plsc.subcore_barrier()
        pltpu.sync_copy(d_src.at[pl.ds(s * SLAB, SLAB)],
                        out_hbm.at[c, 0, pl.ds(s * SLAB, SLAB)])
        pltpu.sync_copy(d_dst.at[pl.ds(s * SLAB, SLAB)],
                        out_hbm.at[c, 1, pl.ds(s * SLAB, SLAB)])

    return kern(src_p, dst_p)


def _sc_edges(y, src_p, dst_p):
    @functools.partial(
        pl.kernel,
        out_type=jax.ShapeDtypeStruct((NC, N_PAD, D), F32),
        mesh=_sc_mesh(),
        scratch_types=[
            pltpu.VMEM((NCH, CW), I32),
            pltpu.VMEM((NCH, CW), I32),
            pltpu.VMEM((CW, D), F32),
            pltpu.VMEM((CW, D), F32),
            pltpu.VMEM_SHARED((N_PAD, D), F32),
            pltpu.SemaphoreType.DMA,
            pltpu.SemaphoreType.DMA,
        ],
    )
    def kern(y_hbm, src_hbm, dst_hbm, out_hbm, src_v, dst_v, ra, rb, acc, sem_a, sem_b):
        c = lax.axis_index("c")
        s = lax.axis_index("s")
        wid = s * NC + c
        pltpu.sync_copy(src_hbm.at[wid], src_v)
        pltpu.sync_copy(dst_hbm.at[wid], dst_v)

        def zfill(i, _):
            for q in range(D // 16):
                ra[i, pl.ds(q * 16, 16)] = jnp.zeros((16,), F32)
            return ()

        lax.fori_loop(0, CW, zfill, ())
        for t in range(SLAB // CW):
            pltpu.sync_copy(ra, acc.at[pl.ds(s * SLAB + t * CW, CW)])
        plsc.subcore_barrier()

        def step(i, _):
            pltpu.async_copy(y_hbm.at[src_v.at[i]], ra, sem_a).wait()
            pltpu.sync_copy(ra, acc.at[dst_v.at[i]], add=True)
            return ()

        lax.fori_loop(0, NCH, step, ())
        plsc.subcore_barrier()
        pltpu.sync_copy(acc.at[pl.ds(s * SLAB, SLAB)],
                        out_hbm.at[c, pl.ds(s * SLAB, SLAB)])

    return kern(y, src_p, dst_p)


def _prep_kern(deg_ref, f_ref, y0_ref, ns_ref, nd_ref):
    d_src = deg_ref[0, 0] + deg_ref[1, 0]
    d_dst = deg_ref[0, 1] + deg_ref[1, 1]
    ns = jnp.where(d_src > 0, lax.rsqrt(d_src), 0.0)
    nd = jnp.where(d_dst > 0, lax.rsqrt(d_dst), 0.0)
    y0_ref[...] = f_ref[...] * ns
    ns_ref[...] = ns
    nd_ref[...] = nd


def _tc_prep(feats_p, degparts):
    deg4 = degparts.reshape(NC, 2, N_PAD, 1)
    grid = N_PAD // 128
    return pl.pallas_call(
        _prep_kern,
        grid=(grid,),
        in_specs=[
            pl.BlockSpec((NC, 2, 128, 1), lambda r: (0, 0, r, 0)),
            pl.BlockSpec((128, D), lambda r: (r, 0)),
        ],
        out_specs=[
            pl.BlockSpec((128, D), lambda r: (r, 0)),
            pl.BlockSpec((128, 1), lambda r: (r, 0)),
            pl.BlockSpec((128, 1), lambda r: (r, 0)),
        ],
        out_shape=[
            jax.ShapeDtypeStruct((N_PAD, D), F32),
            jax.ShapeDtypeStruct((N_PAD, 1), F32),
            jax.ShapeDtypeStruct((N_PAD, 1), F32),
        ],
    )(deg4, feats_p)


def _dense1_kern(agg_ref, nd_ref, w_ref, b_ref, ns_ref, out_ref):
    agg = agg_ref[0] + agg_ref[1]
    x = agg * nd_ref[...]
    h = jnp.dot(x, w_ref[...], preferred_element_type=F32) + b_ref[...]
    out_ref[...] = jnp.maximum(h, 0.0) * ns_ref[...]


def _tc_dense1(aggparts, nd, w, b2d, ns):
    grid = N_PAD // 128
    return pl.pallas_call(
        _dense1_kern,
        grid=(grid,),
        in_specs=[
            pl.BlockSpec((NC, 128, D), lambda r: (0, r, 0)),
            pl.BlockSpec((128, 1), lambda r: (r, 0)),
            pl.BlockSpec((D, D), lambda r: (0, 0)),
            pl.BlockSpec((1, D), lambda r: (0, 0)),
            pl.BlockSpec((128, 1), lambda r: (r, 0)),
        ],
        out_specs=pl.BlockSpec((128, D), lambda r: (r, 0)),
        out_shape=jax.ShapeDtypeStruct((N_PAD, D), F32),
    )(aggparts, nd, w, b2d, ns)


def _dense2_kern(agg_ref, nd_ref, w_ref, b_ref, h_ref, rm_ref):
    agg = agg_ref[0] + agg_ref[1]
    x = agg * nd_ref[...]
    h = jnp.dot(x, w_ref[...], preferred_element_type=F32) + b_ref[...]
    h = jnp.maximum(h, 0.0)
    h_ref[...] = h
    rm = jnp.max(h, axis=1, keepdims=True)
    ridx = pl.program_id(0) * 128 + lax.broadcasted_iota(I32, (128, 1), 0)
    rm_ref[...] = jnp.where(ridx < N, rm, -1.0)


def _tc_dense2(aggparts, nd, w, b2d):
    grid = N_PAD // 128
    return pl.pallas_call(
        _dense2_kern,
        grid=(grid,),
        in_specs=[
            pl.BlockSpec((NC, 128, D), lambda r: (0, r, 0)),
            pl.BlockSpec((128, 1), lambda r: (r, 0)),
            pl.BlockSpec((D, D), lambda r: (0, 0)),
            pl.BlockSpec((1, D), lambda r: (0, 0)),
        ],
        out_specs=[
            pl.BlockSpec((128, D), lambda r: (r, 0)),
            pl.BlockSpec((128, 1), lambda r: (r, 0)),
        ],
        out_shape=[
            jax.ShapeDtypeStruct((N_PAD, D), F32),
            jax.ShapeDtypeStruct((N_PAD, 1), F32),
        ],
    )(aggparts, nd, w, b2d)


def _topk_kern(rm_ref, idx_ref):
    vals0 = rm_ref[...]
    lin = (lax.broadcasted_iota(I32, (80, 128), 0) * 128
           + lax.broadcasted_iota(I32, (80, 128), 1))
    lane = lax.broadcasted_iota(I32, (1, 128), 1)
    big = jnp.int32(2 ** 30)

    def body(k, carry):
        vals, acc = carry
        m = jnp.max(vals)
        cand = jnp.where(vals == m, lin, big)
        j = jnp.min(cand)
        acc = jnp.where(lane == k, j, acc)
        vals = jnp.where(lin == j, -2.0, vals)
        return vals, acc

    _, acc = lax.fori_loop(0, K, body, (vals0, jnp.zeros((1, 128), I32)))
    idx_ref[...] = acc


def _tc_topk(rm2d):
    return pl.pallas_call(
        _topk_kern,
        out_shape=jax.ShapeDtypeStruct((1, 128), I32),
    )(rm2d)


def _gather_kern(idx_ref, h_ref, o_ref):
    o_ref[...] = h_ref[...]


def _tc_gather(idx10, h):
    h3 = h.reshape(N_PAD, 1, D)
    grid_spec = pltpu.PrefetchScalarGridSpec(
        num_scalar_prefetch=1,
        grid=(K,),
        in_specs=[pl.BlockSpec((1, 1, D), lambda k, idx: (idx[k], 0, 0))],
        out_specs=pl.BlockSpec((1, 1, D), lambda k, idx: (k, 0, 0)),
    )
    return pl.pallas_call(
        _gather_kern,
        grid_spec=grid_spec,
        out_shape=jax.ShapeDtypeStruct((K, 1, D), F32),
    )(idx10, h3).reshape(K, D)


def _tail_kern(p_ref, w1_ref, b1_ref, w2_ref, b2_ref, out_ref):
    p = p_ref[...]
    row = lax.broadcasted_iota(I32, (D, D), 0)
    col = lax.broadcasted_iota(I32, (D, D), 1)
    lane = lax.broadcasted_iota(I32, (1, D), 1)
    for k in [2, 4, 8, 16, 32, 64, 128]:
        j = k // 2
        while j >= 1:
            perm_m = (row == jnp.bitwise_xor(col, j)).astype(F32)
            part = jnp.dot(p, perm_m, preferred_element_type=F32,
                           precision=lax.Precision.HIGHEST)
            take_min = ((lane & k) == 0) ^ ((lane & j) != 0)
            p = jnp.where(take_min, jnp.minimum(p, part), jnp.maximum(p, part))
            j //= 2
    u = jnp.dot(p, w1_ref[...], preferred_element_type=F32,
                precision=lax.Precision.HIGHEST) + b1_ref[...]
    u = jnp.maximum(u, 0.0)
    ar = lax.broadcasted_iota(I32, (8, 16), 0)
    ac = lax.broadcasted_iota(I32, (8, 16), 1)
    a_m = (ac == 2 * ar).astype(F32)
    b_m = (ac == 2 * ar + 1).astype(F32)
    v = jnp.maximum(
        jnp.dot(a_m, u, preferred_element_type=F32, precision=lax.Precision.HIGHEST),
        jnp.dot(b_m, u, preferred_element_type=F32, precision=lax.Precision.HIGHEST),
    )
    acc = b2_ref[...]
    for t in range(5):
        acc = acc + jnp.dot(v[t:t + 1, :], w2_ref[t],
                            preferred_element_type=F32,
                            precision=lax.Precision.HIGHEST)
    out_ref[...] = jnp.maximum(acc, 0.0)


def _tc_tail(pooled_p, cw1r, cb1r, cw2m, cb2r):
    return pl.pallas_call(
        _tail_kern,
        out_shape=jax.ShapeDtypeStruct((1, OC), F32),
    )(pooled_p, cw1r, cb1r, cw2m, cb2r)


def kernel(feats, edge_index, W1, b1, W2, b2, cw1, cb1, cw2, cb2):
    pad = NCH * CW - EPW
    es = edge_index.astype(I32).reshape(2, NW, EPW)
    es_p = jnp.concatenate(
        [es, jnp.full((2, NW, pad), TRASH, I32)], axis=2
    ).reshape(2, NW, NCH, CW)
    src_p = es_p[0]
    dst_p = es_p[1]
    feats_p = jnp.pad(feats, ((0, N_PAD - N), (0, 0)))

    degparts = _sc_degrees(src_p, dst_p)
    y0, ns, nd = _tc_prep(feats_p, degparts)

    b1r = b1.reshape(1, D)
    b2r = b2.reshape(1, D)
    agg1 = _sc_edges(y0, src_p, dst_p)
    y1 = _tc_dense1(agg1, nd, W1, b1r, ns)
    agg2 = _sc_edges(y1, src_p, dst_p)
    h2, rm = _tc_dense2(agg2, nd, W2, b2r)

    idxv = _tc_topk(rm.reshape(N_PAD // 128, 128))
    idx10 = idxv[0, :K]
    pooled = _tc_gather(idx10, h2)
    pooled_p = jnp.pad(pooled, ((0, 6), (0, 0)))

    cw1r = jnp.transpose(cw1.reshape(OC // 2, D))
    cb1r = cb1.reshape(1, OC // 2)
    cw2m = jnp.transpose(cw2, (2, 1, 0))
    cb2r = cb2.reshape(1, OC)
    return _tc_tail(pooled_p, cw1r, cb1r, cw2m, cb2r)


# speedup vs baseline: 1.3733x; 1.0044x over previous
import functools

import jax
import jax.numpy as jnp
from jax import lax
from jax.experimental import pallas as pl
from jax.experimental.pallas import tpu as pltpu
from jax.experimental.pallas import tpu_sc as plsc

N = 10000
E = 320000
D = 128
K = 10
OC = 128

NC = 2
NS = 16
NW = NC * NS
N_PAD = 10240
SLAB = N_PAD // NS
CW = 128
NCH = 79
EPW = E // NW
TRASH = N_PAD - 1

F32 = jnp.float32
I32 = jnp.int32


def _sc_mesh():
    return plsc.VectorSubcoreMesh(core_axis_name="c", subcore_axis_name="s")


def _sc_degrees(src_p, dst_p):
    @functools.partial(
        pl.kernel,
        out_type=jax.ShapeDtypeStruct((NC, 2, N_PAD), F32),
        mesh=_sc_mesh(),
        scratch_types=[
            pltpu.VMEM((NCH, CW), I32),
            pltpu.VMEM((NCH, CW), I32),
            pltpu.VMEM((SLAB,), F32),
            pltpu.VMEM((CW,), F32),
            pltpu.VMEM_SHARED((N_PAD,), F32),
            pltpu.VMEM_SHARED((N_PAD,), F32),
        ],
    )
    def kern(src_hbm, dst_hbm, out_hbm, src_v, dst_v, z_v, ones_v, d_src, d_dst):
        c = lax.axis_index("c")
        s = lax.axis_index("s")
        wid = s * NC + c
        pltpu.sync_copy(src_hbm.at[wid], src_v)
        pltpu.sync_copy(dst_hbm.at[wid], dst_v)

        def zfill(i, _):
            z_v[pl.ds(i * 16, 16)] = jnp.zeros((16,), F32)
            return ()

        lax.fori_loop(0, SLAB // 16, zfill, ())
        def ofill(i, _):
            ones_v[pl.ds(i * 16, 16)] = jnp.ones((16,), F32)
            return ()

        lax.fori_loop(0, CW // 16, ofill, ())
        pltpu.sync_copy(z_v, d_src.at[pl.ds(s * SLAB, SLAB)])
        pltpu.sync_copy(z_v, d_dst.at[pl.ds(s * SLAB, SLAB)])
        plsc.subcore_barrier()

        def chunk(i, _):
            pltpu.sync_copy(ones_v, d_src.at[src_v.at[i]], add=True)
            pltpu.sync_copy(ones_v, d_dst.at[dst_v.at[i]], add=True)
            return ()

        lax.fori_loop(0, NCH, chunk, ())
        plsc.subcore_barrier()
        pltpu.sync_copy(d_src.at[pl.ds(s * SLAB, SLAB)],
                        out_hbm.at[c, 0, pl.ds(s * SLAB, SLAB)])
        pltpu.sync_copy(d_dst.at[pl.ds(s * SLAB, SLAB)],
                        out_hbm.at[c, 1, pl.ds(s * SLAB, SLAB)])

    return kern(src_p, dst_p)


def _sc_edges(y, src_p, dst_p):
    @functools.partial(
        pl.kernel,
        out_type=jax.ShapeDtypeStruct((NC, N_PAD, D), F32),
        mesh=_sc_mesh(),
        scratch_types=[
            pltpu.VMEM((NCH, CW), I32),
            pltpu.VMEM((NCH, CW), I32),
            pltpu.VMEM((CW, D), F32),
            pltpu.VMEM((CW, D), F32),
            pltpu.VMEM_SHARED((N_PAD, D), F32),
            pltpu.SemaphoreType.DMA,
            pltpu.SemaphoreType.DMA,
        ],
    )
    def kern(y_hbm, src_hbm, dst_hbm, out_hbm, src_v, dst_v, ra, rb, acc, sem_a, sem_b):
        c = lax.axis_index("c")
        s = lax.axis_index("s")
        wid = s * NC + c
        pltpu.sync_copy(src_hbm.at[wid], src_v)
        pltpu.sync_copy(dst_hbm.at[wid], dst_v)

        def zfill(i, _):
            for q in range(D // 16):
                ra[i, pl.ds(q * 16, 16)] = jnp.zeros((16,), F32)
            return ()

        lax.fori_loop(0, CW, zfill, ())
        for t in range(SLAB // CW):
            pltpu.sync_copy(ra, acc.at[pl.ds(s * SLAB + t * CW, CW)])
        plsc.subcore_barrier()

        def step(i, _):
            pltpu.async_copy(y_hbm.at[src_v.at[i]], ra, sem_a).wait()
            pltpu.sync_copy(ra, acc.at[dst_v.at[i]], add=True)
            return ()

        lax.fori_loop(0, NCH, step, ())
        plsc.subcore_barrier()
        pltpu.sync_copy(acc.at[pl.ds(s * SLAB, SLAB)],
                        out_hbm.at[c, pl.ds(s * SLAB, SLAB)])

    return kern(y, src_p, dst_p)


def _prep_kern(deg_ref, f_ref, y0_ref, ns_ref, nd_ref):
    d_src = deg_ref[0, 0] + deg_ref[1, 0]
    d_dst = deg_ref[0, 1] + deg_ref[1, 1]
    ns = jnp.where(d_src > 0, lax.rsqrt(d_src), 0.0)
    nd = jnp.where(d_dst > 0, lax.rsqrt(d_dst), 0.0)
    y0_ref[...] = f_ref[...] * ns
    ns_ref[...] = ns
    nd_ref[...] = nd


def _tc_prep(feats_p, degparts):
    deg4 = degparts.reshape(NC, 2, N_PAD, 1)
    grid = N_PAD // 128
    return pl.pallas_call(
        _prep_kern,
        grid=(grid,),
        in_specs=[
            pl.BlockSpec((NC, 2, 128, 1), lambda r: (0, 0, r, 0)),
            pl.BlockSpec((128, D), lambda r: (r, 0)),
        ],
        out_specs=[
            pl.BlockSpec((128, D), lambda r: (r, 0)),
            pl.BlockSpec((128, 1), lambda r: (r, 0)),
            pl.BlockSpec((128, 1), lambda r: (r, 0)),
        ],
        out_shape=[
            jax.ShapeDtypeStruct((N_PAD, D), F32),
            jax.ShapeDtypeStruct((N_PAD, 1), F32),
            jax.ShapeDtypeStruct((N_PAD, 1), F32),
        ],
    )(deg4, feats_p)


def _dense1_kern(agg_ref, nd_ref, w_ref, b_ref, ns_ref, out_ref):
    agg = agg_ref[0] + agg_ref[1]
    x = agg * nd_ref[...]
    h = jnp.dot(x, w_ref[...], preferred_element_type=F32) + b_ref[...]
    out_ref[...] = jnp.maximum(h, 0.0) * ns_ref[...]


def _tc_dense1(aggparts, nd, w, b2d, ns):
    grid = N_PAD // 128
    return pl.pallas_call(
        _dense1_kern,
        grid=(grid,),
        in_specs=[
            pl.BlockSpec((NC, 128, D), lambda r: (0, r, 0)),
            pl.BlockSpec((128, 1), lambda r: (r, 0)),
            pl.BlockSpec((D, D), lambda r: (0, 0)),
            pl.BlockSpec((1, D), lambda r: (0, 0)),
            pl.BlockSpec((128, 1), lambda r: (r, 0)),
        ],
        out_specs=pl.BlockSpec((128, D), lambda r: (r, 0)),
        out_shape=jax.ShapeDtypeStruct((N_PAD, D), F32),
    )(aggparts, nd, w, b2d, ns)


def _dense2_kern(agg_ref, nd_ref, w_ref, b_ref, h_ref, rm_ref):
    agg = agg_ref[0] + agg_ref[1]
    x = agg * nd_ref[...]
    h = jnp.dot(x, w_ref[...], preferred_element_type=F32) + b_ref[...]
    h = jnp.maximum(h, 0.0)
    h_ref[...] = h
    rm = jnp.max(h, axis=1, keepdims=True)
    ridx = pl.program_id(0) * 128 + lax.broadcasted_iota(I32, (128, 1), 0)
    rm_ref[...] = jnp.where(ridx < N, rm, -1.0)


def _tc_dense2(aggparts, nd, w, b2d):
    grid = N_PAD // 128
    return pl.pallas_call(
        _dense2_kern,
        grid=(grid,),
        in_specs=[
            pl.BlockSpec((NC, 128, D), lambda r: (0, r, 0)),
            pl.BlockSpec((128, 1), lambda r: (r, 0)),
            pl.BlockSpec((D, D), lambda r: (0, 0)),
            pl.BlockSpec((1, D), lambda r: (0, 0)),
        ],
        out_specs=[
            pl.BlockSpec((128, D), lambda r: (r, 0)),
            pl.BlockSpec((128, 1), lambda r: (r, 0)),
        ],
        out_shape=[
            jax.ShapeDtypeStruct((N_PAD, D), F32),
            jax.ShapeDtypeStruct((N_PAD, 1), F32),
        ],
    )(aggparts, nd, w, b2d)


def _topk_kern(rm_ref, idx_ref):
    vals0 = rm_ref[...]
    lin = (lax.broadcasted_iota(I32, (80, 128), 0) * 128
           + lax.broadcasted_iota(I32, (80, 128), 1))
    lane = lax.broadcasted_iota(I32, (1, 128), 1)
    big = jnp.int32(2 ** 30)

    def body(k, carry):
        vals, acc = carry
        m = jnp.max(vals)
        cand = jnp.where(vals == m, lin, big)
        j = jnp.min(cand)
        acc = jnp.where(lane == k, j, acc)
        vals = jnp.where(lin == j, -2.0, vals)
        return vals, acc

    _, acc = lax.fori_loop(0, K, body, (vals0, jnp.zeros((1, 128), I32)))
    idx_ref[...] = acc


def _tc_topk(rm2d):
    return pl.pallas_call(
        _topk_kern,
        out_shape=jax.ShapeDtypeStruct((1, 128), I32),
    )(rm2d)


def _gather_kern(idx_ref, h_ref, o_ref):
    o_ref[...] = h_ref[...]


def _tc_gather(idx10, h):
    h3 = h.reshape(N_PAD, 1, D)
    grid_spec = pltpu.PrefetchScalarGridSpec(
        num_scalar_prefetch=1,
        grid=(K,),
        in_specs=[pl.BlockSpec((1, 1, D), lambda k, idx: (idx[k], 0, 0))],
        out_specs=pl.BlockSpec((1, 1, D), lambda k, idx: (k, 0, 0)),
    )
    return pl.pallas_call(
        _gather_kern,
        grid_spec=grid_spec,
        out_shape=jax.ShapeDtypeStruct((K, 1, D), F32),
    )(idx10, h3).reshape(K, D)


def _tail_kern(p_ref, w1_ref, b1_ref, w2_ref, b2_ref, out_ref):
    p = p_ref[...]
    row = lax.broadcasted_iota(I32, (D, D), 0)
    col = lax.broadcasted_iota(I32, (D, D), 1)
    lane = lax.broadcasted_iota(I32, (1, D), 1)
    for k in [2, 4, 8, 16, 32, 64, 128]:
        j = k // 2
        while j >= 1:
            perm_m = (row == jnp.bitwise_xor(col, j)).astype(F32)
            part = jnp.dot(p, perm_m, preferred_element_type=F32,
                           precision=lax.Precision.HIGHEST)
            take_min = ((lane & k) == 0) ^ ((lane & j) != 0)
            p = jnp.where(take_min, jnp.minimum(p, part), jnp.maximum(p, part))
            j //= 2
    u = jnp.dot(p, w1_ref[...], preferred_element_type=F32,
                precision=lax.Precision.HIGHEST) + b1_ref[...]
    u = jnp.maximum(u, 0.0)
    ar = lax.broadcasted_iota(I32, (8, 16), 0)
    ac = lax.broadcasted_iota(I32, (8, 16), 1)
    a_m = (ac == 2 * ar).astype(F32)
    b_m = (ac == 2 * ar + 1).astype(F32)
    v = jnp.maximum(
        jnp.dot(a_m, u, preferred_element_type=F32, precision=lax.Precision.HIGHEST),
        jnp.dot(b_m, u, preferred_element_type=F32, precision=lax.Precision.HIGHEST),
    )
    acc = b2_ref[...]
    for t in range(5):
        acc = acc + jnp.dot(v[t:t + 1, :], w2_ref[t],
                            preferred_element_type=F32,
                            precision=lax.Precision.HIGHEST)
    out_ref[...] = jnp.maximum(acc, 0.0)


def _tc_tail(pooled_p, cw1r, cb1r, cw2m, cb2r):
    return pl.pallas_call(
        _tail_kern,
        out_shape=jax.ShapeDtypeStruct((1, OC), F32),
    )(pooled_p, cw1r, cb1r, cw2m, cb2r)


def kernel(feats, edge_index, W1, b1, W2, b2, cw1, cb1, cw2, cb2):
    pad = NCH * CW - EPW
    es = edge_index.astype(I32).reshape(2, NW, EPW)
    wtrash = (N + jnp.arange(NW, dtype=I32))[None, :, None] * jnp.ones(
        (1, 1, pad), I32)
    pad_vals = jnp.concatenate(
        [jnp.full((1, NW, pad), TRASH, I32), wtrash], axis=0)
    es_p = jnp.concatenate([es, pad_vals], axis=2).reshape(2, NW, NCH, CW)
    src_p = es_p[0]
    dst_p = es_p[1]
    feats_p = jnp.pad(feats, ((0, N_PAD - N), (0, 0)))

    degparts = _sc_degrees(src_p, dst_p)
    y0, ns, nd = _tc_prep(feats_p, degparts)

    b1r = b1.reshape(1, D)
    b2r = b2.reshape(1, D)
    agg1 = _sc_edges(y0, src_p, dst_p)
    y1 = _tc_dense1(agg1, nd, W1, b1r, ns)
    agg2 = _sc_edges(y1, src_p, dst_p)
    h2, rm = _tc_dense2(agg2, nd, W2, b2r)

    idxv = _tc_topk(rm.reshape(N_PAD // 128, 128))
    idx10 = idxv[0, :K]
    pooled = _tc_gather(idx10, h2)
    pooled_p = jnp.pad(pooled, ((0, 6), (0, 0)))

    cw1r = jnp.transpose(cw1.reshape(OC // 2, D))
    cb1r = cb1.reshape(1, OC // 2)
    cw2m = jnp.transpose(cw2, (2, 1, 0))
    cb2r = cb2.reshape(1, OC)
    return _tc_tail(pooled_p, cw1r, cb1r, cw2m, cb2r)
